# baseline (speedup 1.0000x reference)
import jax
import jax.numpy as jnp
from jax import lax
from jax.experimental import pallas as pl
from jax.experimental.pallas import tpu as pltpu
from jax.experimental.pallas import tpu_sc as plsc

N_ROWS = 50000
D = 32
E = 1600000
NUM_CORES = 2
NUM_SUBCORES = 16
LANES = 16

CHUNK = 256
SUB = 128
NSUB = CHUNK // SUB
GROUP = 1024
GROWS = GROUP // SUB
CPG = GROUP // CHUNK
PER_TILE = 100352
E_PAD = PER_TILE * NUM_SUBCORES
NGRP = PER_TILE // GROUP
NCHUNK = PER_TILE // CHUNK

N_PAD = 50048
ROWS_PER_TILE = N_PAD // NUM_SUBCORES


def _process_spmm(sid, rows_hbm, cols_hbm, vals_hbm, emb_hbm, out_hbm,
                  rows_m, cols_m, vals_m, rows_v, acc, gsems, msems,
                  ssems):
    grp0 = sid * NGRP

    def issue_meta(G, r):
        blk = (grp0 + G) * GROWS
        pltpu.async_copy(rows_hbm.at[pl.ds(blk, GROWS)], rows_m.at[r],
                         msems[r])
        pltpu.async_copy(cols_hbm.at[pl.ds(blk, GROWS)], cols_m.at[r],
                         msems[r])
        pltpu.async_copy(vals_hbm.at[pl.ds(blk, GROWS)], vals_m.at[r],
                         msems[r])

    def wait_meta(G, r):
        blk = (grp0 + G) * GROWS
        pltpu.make_async_copy(rows_hbm.at[pl.ds(blk, GROWS)], rows_m.at[r],
                              msems[r]).wait()
        pltpu.make_async_copy(cols_hbm.at[pl.ds(blk, GROWS)], cols_m.at[r],
                              msems[r]).wait()
        pltpu.make_async_copy(vals_hbm.at[pl.ds(blk, GROWS)], vals_m.at[r],
                              msems[r]).wait()

    def issue_gather(r, k, b):
        for j in range(NSUB):
            pltpu.async_copy(emb_hbm.at[cols_m.at[r, NSUB * k + j]],
                             rows_v.at[b, pl.ds(j * SUB, SUB)], gsems[b])

    def wait_gather(r, k, b):
        for j in range(NSUB):
            pltpu.make_async_copy(emb_hbm.at[cols_m.at[r, NSUB * k + j]],
                                  rows_v.at[b, pl.ds(j * SUB, SUB)],
                                  gsems[b]).wait()

    def scale(r, k, b):
        @plsc.parallel_loop(0, CHUNK // LANES, unroll=4)
        def _(grp):
            vv = vals_m[r, NSUB * k + (grp >> 3),
                        pl.ds((grp & 7) * LANES, LANES)]
            e0 = grp * LANES
            for i in range(LANES):
                v = vv[i]
                r0 = rows_v[b, e0 + i, pl.ds(0, LANES)]
                rows_v[b, e0 + i, pl.ds(0, LANES)] = r0 * v
                r1 = rows_v[b, e0 + i, pl.ds(LANES, LANES)]
                rows_v[b, e0 + i, pl.ds(LANES, LANES)] = r1 * v

    def issue_scatter(r, k, b):
        for j in range(NSUB):
            pltpu.async_copy(rows_v.at[b, pl.ds(j * SUB, SUB)],
                             acc.at[rows_m.at[r, NSUB * k + j]], ssems[b],
                             add=True)

    def wait_scatter(b):
        for j in range(NSUB):
            pltpu.make_async_copy(rows_v.at[b, pl.ds(j * SUB, SUB)],
                                  acc.at[rows_m.at[0, j]], ssems[b]).wait()

    def zero_body(i, _):
        rows_v[0, i >> 1, pl.ds((i & 1) * LANES, LANES)] = jnp.zeros(
            (LANES,), jnp.float32)
        return 0
    lax.fori_loop(0, 2 * CHUNK, zero_body, 0)
    row0 = sid * ROWS_PER_TILE
    nfull = ROWS_PER_TILE // CHUNK
    rem = ROWS_PER_TILE - nfull * CHUNK
    for k in range(nfull):
        pltpu.sync_copy(rows_v.at[0, pl.ds(0, CHUNK)],
                        acc.at[pl.ds(row0 + k * CHUNK, CHUNK)])
    pltpu.sync_copy(rows_v.at[0, pl.ds(0, rem)],
                    acc.at[pl.ds(row0 + nfull * CHUNK, rem)])
    plsc.subcore_barrier()

    issue_meta(0, 0)
    wait_meta(0, 0)
    issue_gather(0, 0, 0)

    def pair_body(i, _):
        for r in (0, 1):
            G = 2 * i + r

            @pl.when(G + 1 < NGRP)
            def _():
                issue_meta(G + 1, 1 - r)

            for k in range(CPG):
                b = k & 1
                wait_gather(r, k, b)
                scale(r, k, b)
                if r == 0 and k == 0:
                    @pl.when(i > 0)
                    def _():
                        wait_scatter(1 - b)
                else:
                    wait_scatter(1 - b)
                if k < CPG - 1:
                    issue_gather(r, k + 1, 1 - b)
                else:
                    @pl.when(G + 1 < NGRP)
                    def _():
                        wait_meta(G + 1, 1 - r)
                        issue_gather(1 - r, 0, 1 - b)
                issue_scatter(r, k, b)
        return 0
    lax.fori_loop(0, NGRP // 2, pair_body, 0)
    wait_scatter(1)

    plsc.subcore_barrier()

    pltpu.sync_copy(acc.at[pl.ds(row0, ROWS_PER_TILE)],
                    out_hbm.at[pl.ds(row0, ROWS_PER_TILE)])


def _sc_kernel(u2i_r, u2i_c, u2i_v, i2u_r, i2u_c, i2u_v,
               user_emb, item_emb, out_user, out_item,
               rows_m, cols_m, vals_m, rows_v, acc, g0, g1, m0, m1, s0, s1):
    cid = lax.axis_index("c")
    sid = lax.axis_index("s")
    gsems = (g0, g1)
    msems = (m0, m1)
    ssems = (s0, s1)

    @pl.when(cid == 0)
    def _():
        _process_spmm(sid, u2i_r, u2i_c, u2i_v, item_emb, out_user,
                      rows_m, cols_m, vals_m, rows_v, acc, gsems, msems,
                      ssems)

    @pl.when(cid == 1)
    def _():
        _process_spmm(sid, i2u_r, i2u_c, i2u_v, user_emb, out_item,
                      rows_m, cols_m, vals_m, rows_v, acc, gsems, msems,
                      ssems)


@jax.jit
def _lightgcn(user_embedding, item_embedding, u2i_indices, u2i_values,
              i2u_indices, i2u_values):
    pad = E_PAD - E

    def prep(indices, values):
        rows = jnp.pad(indices[0].astype(jnp.int32),
                       (0, pad)).reshape(E_PAD // SUB, SUB)
        cols = jnp.pad(indices[1].astype(jnp.int32),
                       (0, pad)).reshape(E_PAD // SUB, SUB)
        vals = jnp.pad(values.astype(jnp.float32),
                       (0, pad)).reshape(E_PAD // SUB, SUB)
        return rows, cols, vals

    u2i_r, u2i_c, u2i_v = prep(u2i_indices, u2i_values)
    i2u_r, i2u_c, i2u_v = prep(i2u_indices, i2u_values)

    mesh = plsc.VectorSubcoreMesh(core_axis_name="c", subcore_axis_name="s")
    run = pl.kernel(
        _sc_kernel,
        out_type=(
            jax.ShapeDtypeStruct((N_PAD, D), jnp.float32),
            jax.ShapeDtypeStruct((N_PAD, D), jnp.float32),
        ),
        mesh=mesh,
        scratch_types=[
            pltpu.VMEM((2, GROWS, SUB), jnp.int32),
            pltpu.VMEM((2, GROWS, SUB), jnp.int32),
            pltpu.VMEM((2, GROWS, SUB), jnp.float32),
            pltpu.VMEM((2, CHUNK, D), jnp.float32),
            pltpu.VMEM_SHARED((N_PAD, D), jnp.float32),
            pltpu.SemaphoreType.DMA,
            pltpu.SemaphoreType.DMA,
            pltpu.SemaphoreType.DMA,
            pltpu.SemaphoreType.DMA,
            pltpu.SemaphoreType.DMA,
            pltpu.SemaphoreType.DMA,
        ],
        compiler_params=pltpu.CompilerParams(use_tc_tiling_on_sc=False,
                                             needs_layout_passes=False),
    )
    out_user, out_item = run(u2i_r, u2i_c, u2i_v, i2u_r, i2u_c, i2u_v,
                             user_embedding, item_embedding)
    return out_user[:N_ROWS], out_item[:N_ROWS]


def kernel(user_embedding, item_embedding, u2i_indices, u2i_values,
           i2u_indices, i2u_values):
    return _lightgcn(user_embedding, item_embedding, u2i_indices, u2i_values,
                     i2u_indices, i2u_values)


# speedup vs baseline: 1.1953x; 1.1953x over previous
import jax
import jax.numpy as jnp
from jax import lax
from jax.experimental import pallas as pl
from jax.experimental.pallas import tpu as pltpu
from jax.experimental.pallas import tpu_sc as plsc

N_ROWS = 50000
D = 32
E = 1600000
NUM_CORES = 2
NUM_SUBCORES = 16
LANES = 16

CHUNK = 256
SUB = 128
NSUB = CHUNK // SUB
GROUP = 1024
GROWS = GROUP // SUB
CPG = GROUP // CHUNK
PER_TILE = 100352
E_PAD = PER_TILE * NUM_SUBCORES
NGRP = PER_TILE // GROUP
NCHUNK = PER_TILE // CHUNK

N_PAD = 50048
ROWS_PER_TILE = N_PAD // NUM_SUBCORES


def _process_spmm(sid, rows_hbm, cols_hbm, vals_hbm, emb_hbm, out_hbm,
                  rows_m, cols_m, vals_m, rows_v, acc, gsems, msems):
    grp0 = sid * NGRP

    def issue_meta(G, r):
        blk = (grp0 + G) * GROWS
        pltpu.async_copy(rows_hbm.at[pl.ds(blk, GROWS)], rows_m.at[r],
                         msems[r])
        pltpu.async_copy(cols_hbm.at[pl.ds(blk, GROWS)], cols_m.at[r],
                         msems[r])
        pltpu.async_copy(vals_hbm.at[pl.ds(blk, GROWS)], vals_m.at[r],
                         msems[r])

    def wait_meta(G, r):
        blk = (grp0 + G) * GROWS
        pltpu.make_async_copy(rows_hbm.at[pl.ds(blk, GROWS)], rows_m.at[r],
                              msems[r]).wait()
        pltpu.make_async_copy(cols_hbm.at[pl.ds(blk, GROWS)], cols_m.at[r],
                              msems[r]).wait()
        pltpu.make_async_copy(vals_hbm.at[pl.ds(blk, GROWS)], vals_m.at[r],
                              msems[r]).wait()

    def issue_gather(r, k, b):
        for j in range(NSUB):
            pltpu.async_copy(emb_hbm.at[cols_m.at[r, NSUB * k + j]],
                             rows_v.at[b, pl.ds(j * SUB, SUB)], gsems[b])

    def wait_gather(r, k, b):
        for j in range(NSUB):
            pltpu.make_async_copy(emb_hbm.at[cols_m.at[r, NSUB * k + j]],
                                  rows_v.at[b, pl.ds(j * SUB, SUB)],
                                  gsems[b]).wait()

    def scale(r, k, b):
        @plsc.parallel_loop(0, CHUNK // LANES, unroll=4)
        def _(grp):
            vv = vals_m[r, NSUB * k + (grp >> 3),
                        pl.ds((grp & 7) * LANES, LANES)]
            e0 = grp * LANES
            for i in range(LANES):
                v = vv[i]
                r0 = rows_v[b, e0 + i, pl.ds(0, LANES)]
                rows_v[b, e0 + i, pl.ds(0, LANES)] = r0 * v
                r1 = rows_v[b, e0 + i, pl.ds(LANES, LANES)]
                rows_v[b, e0 + i, pl.ds(LANES, LANES)] = r1 * v

    def scatter(r, k, b):
        for j in range(NSUB):
            pltpu.sync_copy(rows_v.at[b, pl.ds(j * SUB, SUB)],
                            acc.at[rows_m.at[r, NSUB * k + j]], add=True)

    def zero_body(i, _):
        rows_v[0, i >> 1, pl.ds((i & 1) * LANES, LANES)] = jnp.zeros(
            (LANES,), jnp.float32)
        return 0
    lax.fori_loop(0, 2 * CHUNK, zero_body, 0)
    row0 = sid * ROWS_PER_TILE
    nfull = ROWS_PER_TILE // CHUNK
    rem = ROWS_PER_TILE - nfull * CHUNK
    for k in range(nfull):
        pltpu.sync_copy(rows_v.at[0, pl.ds(0, CHUNK)],
                        acc.at[pl.ds(row0 + k * CHUNK, CHUNK)])
    pltpu.sync_copy(rows_v.at[0, pl.ds(0, rem)],
                    acc.at[pl.ds(row0 + nfull * CHUNK, rem)])
    plsc.subcore_barrier()

    issue_meta(0, 0)
    wait_meta(0, 0)
    issue_gather(0, 0, 0)

    def pair_body(i, _):
        for r in (0, 1):
            G = 2 * i + r

            @pl.when(G + 1 < NGRP)
            def _():
                issue_meta(G + 1, 1 - r)

            for k in range(CPG):
                b = k & 1
                if k < CPG - 1:
                    wait_gather(r, k, b)
                    issue_gather(r, k + 1, 1 - b)
                else:
                    @pl.when(G + 1 < NGRP)
                    def _():
                        wait_meta(G + 1, 1 - r)
                        issue_gather(1 - r, 0, 1 - b)
                    wait_gather(r, k, b)
                scale(r, k, b)
                scatter(r, k, b)
        return 0
    lax.fori_loop(0, NGRP // 2, pair_body, 0)

    plsc.subcore_barrier()

    pltpu.sync_copy(acc.at[pl.ds(row0, ROWS_PER_TILE)],
                    out_hbm.at[pl.ds(row0, ROWS_PER_TILE)])


def _sc_kernel(u2i_r, u2i_c, u2i_v, i2u_r, i2u_c, i2u_v,
               user_emb, item_emb, out_user, out_item,
               rows_m, cols_m, vals_m, rows_v, acc, g0, g1, m0, m1):
    cid = lax.axis_index("c")
    sid = lax.axis_index("s")
    gsems = (g0, g1)
    msems = (m0, m1)

    @pl.when(cid == 0)
    def _():
        _process_spmm(sid, u2i_r, u2i_c, u2i_v, item_emb, out_user,
                      rows_m, cols_m, vals_m, rows_v, acc, gsems, msems)

    @pl.when(cid == 1)
    def _():
        _process_spmm(sid, i2u_r, i2u_c, i2u_v, user_emb, out_item,
                      rows_m, cols_m, vals_m, rows_v, acc, gsems, msems)


@jax.jit
def _lightgcn(user_embedding, item_embedding, u2i_indices, u2i_values,
              i2u_indices, i2u_values):
    pad = E_PAD - E

    def prep(indices, values):
        rows = jnp.pad(indices[0].astype(jnp.int32),
                       (0, pad)).reshape(E_PAD // SUB, SUB)
        cols = jnp.pad(indices[1].astype(jnp.int32),
                       (0, pad)).reshape(E_PAD // SUB, SUB)
        vals = jnp.pad(values.astype(jnp.float32),
                       (0, pad)).reshape(E_PAD // SUB, SUB)
        return rows, cols, vals

    u2i_r, u2i_c, u2i_v = prep(u2i_indices, u2i_values)
    i2u_r, i2u_c, i2u_v = prep(i2u_indices, i2u_values)

    mesh = plsc.VectorSubcoreMesh(core_axis_name="c", subcore_axis_name="s")
    run = pl.kernel(
        _sc_kernel,
        out_type=(
            jax.ShapeDtypeStruct((N_PAD, D), jnp.float32),
            jax.ShapeDtypeStruct((N_PAD, D), jnp.float32),
        ),
        mesh=mesh,
        scratch_types=[
            pltpu.VMEM((2, GROWS, SUB), jnp.int32),
            pltpu.VMEM((2, GROWS, SUB), jnp.int32),
            pltpu.VMEM((2, GROWS, SUB), jnp.float32),
            pltpu.VMEM((2, CHUNK, D), jnp.float32),
            pltpu.VMEM_SHARED((N_PAD, D), jnp.float32),
            pltpu.SemaphoreType.DMA,
            pltpu.SemaphoreType.DMA,
            pltpu.SemaphoreType.DMA,
            pltpu.SemaphoreType.DMA,
        ],
        compiler_params=pltpu.CompilerParams(use_tc_tiling_on_sc=False,
                                             needs_layout_passes=False),
    )
    out_user, out_item = run(u2i_r, u2i_c, u2i_v, i2u_r, i2u_c, i2u_v,
                             user_embedding, item_embedding)
    return out_user[:N_ROWS], out_item[:N_ROWS]


def kernel(user_embedding, item_embedding, u2i_indices, u2i_values,
           i2u_indices, i2u_values):
    return _lightgcn(user_embedding, item_embedding, u2i_indices, u2i_values,
                     i2u_indices, i2u_values)


# speedup vs baseline: 1.3743x; 1.1497x over previous
import jax
import jax.numpy as jnp
from jax import lax
from jax.experimental import pallas as pl
from jax.experimental.pallas import tpu as pltpu
from jax.experimental.pallas import tpu_sc as plsc

N_ROWS = 50000
D = 32
E = 1600000
NUM_CORES = 2
NUM_SUBCORES = 16
LANES = 16

CHUNK = 256
SUB = 128
NSUB = CHUNK // SUB
GROUP = 1024
GROWS = GROUP // SUB
CPG = GROUP // CHUNK

MAIN_GRP = 96
MAIN_PER_TILE = MAIN_GRP * GROUP
MAIN_TOT = MAIN_PER_TILE * NUM_SUBCORES
TAIL_REAL = (E - MAIN_TOT) // NUM_SUBCORES
TAIL_GRP = 2
TAIL_PER_TILE = TAIL_GRP * GROUP
TAIL_TOT = TAIL_PER_TILE * NUM_SUBCORES

N_PAD = 50048
ROWS_PER_TILE = N_PAD // NUM_SUBCORES


def _process_spmm(sid, idx_hbm, vals_hbm, tidx_hbm, tvals_hbm, emb_hbm,
                  out_hbm, idx_m, vals_m, rows_v, acc, gsems, msems):
    main0 = sid * MAIN_PER_TILE
    tail0 = sid * TAIL_PER_TILE

    def issue_meta(src_idx, src_vals, e0, r):
        for s in range(GROWS):
            pltpu.async_copy(src_idx.at[:, pl.ds(e0 + s * SUB, SUB)],
                             idx_m.at[r, s], msems[r])
        pltpu.async_copy(src_vals.at[pl.ds(e0, GROUP)], vals_m.at[r],
                         msems[r])

    def wait_meta(src_idx, src_vals, e0, r):
        for s in range(GROWS):
            pltpu.make_async_copy(src_idx.at[:, pl.ds(e0 + s * SUB, SUB)],
                                  idx_m.at[r, s], msems[r]).wait()
        pltpu.make_async_copy(src_vals.at[pl.ds(e0, GROUP)], vals_m.at[r],
                              msems[r]).wait()

    def issue_gather(r, k, b):
        for j in range(NSUB):
            pltpu.async_copy(emb_hbm.at[idx_m.at[r, NSUB * k + j, 1]],
                             rows_v.at[b, pl.ds(j * SUB, SUB)], gsems[b])

    def wait_gather(r, k, b):
        for j in range(NSUB):
            pltpu.make_async_copy(emb_hbm.at[idx_m.at[r, NSUB * k + j, 1]],
                                  rows_v.at[b, pl.ds(j * SUB, SUB)],
                                  gsems[b]).wait()

    def scale(r, k, b):
        @plsc.parallel_loop(0, CHUNK // LANES, unroll=2)
        def _(grp):
            vv = vals_m[r, pl.ds(k * CHUNK + grp * LANES, LANES)]
            e0 = grp * LANES
            for i in range(LANES):
                v = vv[i]
                r0 = rows_v[b, e0 + i, pl.ds(0, LANES)]
                rows_v[b, e0 + i, pl.ds(0, LANES)] = r0 * v
                r1 = rows_v[b, e0 + i, pl.ds(LANES, LANES)]
                rows_v[b, e0 + i, pl.ds(LANES, LANES)] = r1 * v

    def scatter(r, k, b):
        for j in range(NSUB):
            pltpu.sync_copy(rows_v.at[b, pl.ds(j * SUB, SUB)],
                            acc.at[idx_m.at[r, NSUB * k + j, 0]], add=True)

    def zero_body(i, _):
        rows_v[0, i >> 1, pl.ds((i & 1) * LANES, LANES)] = jnp.zeros(
            (LANES,), jnp.float32)
        return 0
    lax.fori_loop(0, 2 * CHUNK, zero_body, 0)
    row0 = sid * ROWS_PER_TILE
    nfull = ROWS_PER_TILE // CHUNK
    rem = ROWS_PER_TILE - nfull * CHUNK
    for k in range(nfull):
        pltpu.sync_copy(rows_v.at[0, pl.ds(0, CHUNK)],
                        acc.at[pl.ds(row0 + k * CHUNK, CHUNK)])
    pltpu.sync_copy(rows_v.at[0, pl.ds(0, rem)],
                    acc.at[pl.ds(row0 + nfull * CHUNK, rem)])
    plsc.subcore_barrier()

    issue_meta(idx_hbm, vals_hbm, main0, 0)
    wait_meta(idx_hbm, vals_hbm, main0, 0)
    issue_gather(0, 0, 0)

    def pair_body(i, _):
        for r in (0, 1):
            G = 2 * i + r
            e_next = main0 + (G + 1) * GROUP

            @pl.when(G + 1 < MAIN_GRP)
            def _():
                issue_meta(idx_hbm, vals_hbm, e_next, 1 - r)

            for k in range(CPG):
                b = k & 1
                if k < CPG - 1:
                    wait_gather(r, k, b)
                    issue_gather(r, k + 1, 1 - b)
                else:
                    @pl.when(G + 1 < MAIN_GRP)
                    def _():
                        wait_meta(idx_hbm, vals_hbm, e_next, 1 - r)
                        issue_gather(1 - r, 0, 1 - b)
                    wait_gather(r, k, b)
                scale(r, k, b)
                scatter(r, k, b)
        return 0
    lax.fori_loop(0, MAIN_GRP // 2, pair_body, 0)

    for tg in range(TAIL_GRP):
        e0 = tail0 + tg * GROUP
        issue_meta(tidx_hbm, tvals_hbm, e0, tg)
        wait_meta(tidx_hbm, tvals_hbm, e0, tg)
        issue_gather(tg, 0, 0)
        for k in range(CPG):
            b = k & 1
            if k < CPG - 1:
                wait_gather(tg, k, b)
                issue_gather(tg, k + 1, 1 - b)
            else:
                wait_gather(tg, k, b)
            scale(tg, k, b)
            scatter(tg, k, b)

    plsc.subcore_barrier()

    pltpu.sync_copy(acc.at[pl.ds(row0, ROWS_PER_TILE)],
                    out_hbm.at[pl.ds(row0, ROWS_PER_TILE)])


def _sc_kernel(u2i_i, u2i_v, u2i_ti, u2i_tv, i2u_i, i2u_v, i2u_ti, i2u_tv,
               user_emb, item_emb, out_user, out_item,
               idx_m, vals_m, rows_v, acc, g0, g1, m0, m1):
    cid = lax.axis_index("c")
    sid = lax.axis_index("s")
    gsems = (g0, g1)
    msems = (m0, m1)

    @pl.when(cid == 0)
    def _():
        _process_spmm(sid, u2i_i, u2i_v, u2i_ti, u2i_tv, item_emb, out_user,
                      idx_m, vals_m, rows_v, acc, gsems, msems)

    @pl.when(cid == 1)
    def _():
        _process_spmm(sid, i2u_i, i2u_v, i2u_ti, i2u_tv, user_emb, out_item,
                      idx_m, vals_m, rows_v, acc, gsems, msems)


@jax.jit
def _lightgcn(user_embedding, item_embedding, u2i_indices, u2i_values,
              i2u_indices, i2u_values):
    def prep(indices, values):
        idx = indices.astype(jnp.int32)
        vals = values.astype(jnp.float32)
        tidx = idx[:, MAIN_TOT:].reshape(2, NUM_SUBCORES, TAIL_REAL)
        tidx = jnp.pad(tidx, ((0, 0), (0, 0), (0, TAIL_PER_TILE - TAIL_REAL)))
        tvals = vals[MAIN_TOT:].reshape(NUM_SUBCORES, TAIL_REAL)
        tvals = jnp.pad(tvals, ((0, 0), (0, TAIL_PER_TILE - TAIL_REAL)))
        return (idx, vals, tidx.reshape(2, TAIL_TOT),
                tvals.reshape(TAIL_TOT))

    u2i_i, u2i_v, u2i_ti, u2i_tv = prep(u2i_indices, u2i_values)
    i2u_i, i2u_v, i2u_ti, i2u_tv = prep(i2u_indices, i2u_values)

    mesh = plsc.VectorSubcoreMesh(core_axis_name="c", subcore_axis_name="s")
    run = pl.kernel(
        _sc_kernel,
        out_type=(
            jax.ShapeDtypeStruct((N_PAD, D), jnp.float32),
            jax.ShapeDtypeStruct((N_PAD, D), jnp.float32),
        ),
        mesh=mesh,
        scratch_types=[
            pltpu.VMEM((2, GROWS, 2, SUB), jnp.int32),
            pltpu.VMEM((2, GROUP), jnp.float32),
            pltpu.VMEM((2, CHUNK, D), jnp.float32),
            pltpu.VMEM_SHARED((N_PAD, D), jnp.float32),
            pltpu.SemaphoreType.DMA,
            pltpu.SemaphoreType.DMA,
            pltpu.SemaphoreType.DMA,
            pltpu.SemaphoreType.DMA,
        ],
        compiler_params=pltpu.CompilerParams(use_tc_tiling_on_sc=False,
                                             needs_layout_passes=False),
    )
    out_user, out_item = run(u2i_i, u2i_v, u2i_ti, u2i_tv,
                             i2u_i, i2u_v, i2u_ti, i2u_tv,
                             user_embedding, item_embedding)
    return out_user[:N_ROWS], out_item[:N_ROWS]


def kernel(user_embedding, item_embedding, u2i_indices, u2i_values,
           i2u_indices, i2u_values):
    return _lightgcn(user_embedding, item_embedding, u2i_indices, u2i_values,
                     i2u_indices, i2u_values)
